# baseline (speedup 1.0000x reference)
import functools

import jax
import jax.numpy as jnp
from jax import lax
from jax.experimental import pallas as pl
from jax.experimental.pallas import tpu as pltpu
from jax.experimental.pallas import tpu_sc as plsc

N = 10000
P = 10240
DI = 128
DO = 64
E = 320000
NC = 2
NS = 16
NTILES = NC * NS
CHUNK = 128
CPT0 = 128
CPT1 = 32
EP = NS * (CPT0 + CPT1) * CHUNK
ROWS0 = NS * CPT0
SENT = N
CNT_ROWS = 256
RPT = P // NS
ZROWS = 160
NBUF = 4

_sc_mesh = plsc.VectorSubcoreMesh(
    core_axis_name="c", subcore_axis_name="s", num_cores=NC, num_subcores=NS
)


def _deg_body(src_hbm, dst_hbm, out_hbm, idx_s, idx_d, cnt_s, cnt_d):
    c = lax.axis_index("c")
    s = lax.axis_index("s")
    tile = c * NS + s
    nch = jnp.where(c == 0, CPT0, CPT1)
    base = jnp.where(c == 0, s * CPT0, ROWS0 + s * CPT1)
    zero16 = jnp.zeros((16,), jnp.float32)
    ones16 = jnp.ones((16,), jnp.float32)

    def zloop(r, carry):
        for cc in range(8):
            cnt_s[r, pl.ds(cc * 16, 16)] = zero16
            cnt_d[r, pl.ds(cc * 16, 16)] = zero16
        return carry

    lax.fori_loop(0, 128, zloop, 0)

    def lloop(i, carry):
        pltpu.sync_copy(src_hbm.at[pl.ds(base + i * 32, 32)],
                        idx_s.at[pl.ds(i * 32, 32)])
        pltpu.sync_copy(dst_hbm.at[pl.ds(base + i * 32, 32)],
                        idx_d.at[pl.ds(i * 32, 32)])
        return carry

    lax.fori_loop(0, nch // 32, lloop, 0)

    def cloop(r, carry):
        for cc in range(8):
            v = idx_s[r, pl.ds(cc * 16, 16)]
            plsc.addupdate_scatter(cnt_s, [v >> 7, v & 127], ones16)
            w = idx_d[r, pl.ds(cc * 16, 16)]
            plsc.addupdate_scatter(cnt_d, [w >> 7, w & 127], ones16)
        return carry

    lax.fori_loop(0, nch, cloop, 0)

    pltpu.sync_copy(cnt_s, out_hbm.at[0, tile])
    pltpu.sync_copy(cnt_d, out_hbm.at[1, tile])


_deg = pl.kernel(
    _deg_body,
    out_type=jax.ShapeDtypeStruct((2, NTILES, 128, 128), jnp.float32),
    mesh=_sc_mesh,
    scratch_types=[
        pltpu.VMEM((CPT0, CHUNK), jnp.int32),
        pltpu.VMEM((CPT0, CHUNK), jnp.int32),
        pltpu.VMEM((128, 128), jnp.float32),
        pltpu.VMEM((128, 128), jnp.float32),
    ],
    compiler_params=pltpu.CompilerParams(needs_layout_passes=False),
)


CPTT = EP // CHUNK // NS
PH = 2
CPP = CPTT // PH


def _prop_body(h_hbm, src_hbm, dst_hbm, out_hbm, idx_s, idx_d, rows, zbuf,
               acc, gsem, ssem):
    c = lax.axis_index("c")
    s = lax.axis_index("s")

    @pl.when(c == 0)
    def _():
        zero16 = jnp.zeros((16,), jnp.float32)

        def zloop(r, carry):
            for cc in range(4):
                zbuf[r, pl.ds(cc * 16, 16)] = zero16
            return carry

        lax.fori_loop(0, ZROWS, zloop, 0)
        for k in range(RPT // ZROWS):
            pltpu.sync_copy(zbuf, acc.at[pl.ds(s * RPT + k * ZROWS, ZROWS)])

        plsc.subcore_barrier()

        def gloop(g, carry):
            g0 = g * NBUF
            cps = [
                pltpu.async_copy(h_hbm.at[idx_s.at[g0 + b]], rows.at[b], gsem)
                for b in range(NBUF)
            ]
            for cp in cps:
                cp.wait()
            cps = [
                pltpu.async_copy(rows.at[b], acc.at[idx_d.at[g0 + b]], ssem,
                                 add=True)
                for b in range(NBUF)
            ]
            for cp in cps:
                cp.wait()
            return carry

        for ph in range(PH):
            pltpu.sync_copy(src_hbm.at[pl.ds(s * CPTT + ph * CPP, CPP)], idx_s)
            pltpu.sync_copy(dst_hbm.at[pl.ds(s * CPTT + ph * CPP, CPP)], idx_d)
            lax.fori_loop(0, CPP // NBUF, gloop, 0)

        plsc.subcore_barrier()
        pltpu.sync_copy(acc.at[pl.ds(s * RPT, RPT)],
                        out_hbm.at[pl.ds(s * RPT, RPT)])


_prop = pl.kernel(
    _prop_body,
    out_type=jax.ShapeDtypeStruct((P, DO), jnp.float32),
    mesh=_sc_mesh,
    scratch_types=[
        pltpu.VMEM((CPP, CHUNK), jnp.int32),
        pltpu.VMEM((CPP, CHUNK), jnp.int32),
        pltpu.VMEM((NBUF, CHUNK, DO), jnp.float32),
        pltpu.VMEM((ZROWS, DO), jnp.float32),
        pltpu.VMEM_SHARED((P, DO), jnp.float32),
        pltpu.SemaphoreType.DMA,
        pltpu.SemaphoreType.DMA,
    ],
    compiler_params=pltpu.CompilerParams(
        needs_layout_passes=False, use_tc_tiling_on_sc=False
    ),
)

BR = 1280
G = P // BR


def _k1_body(x_ref, w_ref, cs_ref, cd_ref, h0_ref, ns_ref, nd_ref):
    cs = jnp.sum(cs_ref[...], axis=0)
    cd = jnp.sum(cd_ref[...], axis=0)
    ns = lax.rsqrt(jnp.maximum(cs, 1.0))
    nd = lax.rsqrt(jnp.maximum(cd, 1.0))
    ns_ref[0, :] = ns
    nd_ref[0, :] = nd
    h0_ref[...] = jnp.dot(x_ref[...], w_ref[...],
                          preferred_element_type=jnp.float32) * ns[:, None]


_k1 = pl.pallas_call(
    _k1_body,
    grid=(G,),
    in_specs=[
        pl.BlockSpec((BR, DI), lambda i: (i, 0)),
        pl.BlockSpec((DI, DO), lambda i: (0, 0)),
        pl.BlockSpec((NTILES, BR), lambda i: (0, i)),
        pl.BlockSpec((NTILES, BR), lambda i: (0, i)),
    ],
    out_specs=[
        pl.BlockSpec((BR, DO), lambda i: (i, 0)),
        pl.BlockSpec((1, BR), lambda i: (0, i)),
        pl.BlockSpec((1, BR), lambda i: (0, i)),
    ],
    out_shape=[
        jax.ShapeDtypeStruct((P, DO), jnp.float32),
        jax.ShapeDtypeStruct((1, P), jnp.float32),
        jax.ShapeDtypeStruct((1, P), jnp.float32),
    ],
)


def _k2_body(p_ref, ns_ref, nd_ref, w_ref, b_ref, out_ref):
    agg = p_ref[...] * nd_ref[0, :][:, None]
    h = jnp.tanh(agg + b_ref[0, :][None, :])
    out_ref[...] = jnp.dot(h, w_ref[...],
                           preferred_element_type=jnp.float32) * ns_ref[0, :][:, None]


_k2 = pl.pallas_call(
    _k2_body,
    grid=(G,),
    in_specs=[
        pl.BlockSpec((BR, DO), lambda i: (i, 0)),
        pl.BlockSpec((1, BR), lambda i: (0, i)),
        pl.BlockSpec((1, BR), lambda i: (0, i)),
        pl.BlockSpec((DO, DO), lambda i: (0, 0)),
        pl.BlockSpec((1, DO), lambda i: (0, 0)),
    ],
    out_specs=pl.BlockSpec((BR, DO), lambda i: (i, 0)),
    out_shape=jax.ShapeDtypeStruct((P, DO), jnp.float32),
)


def _k3_body(p_ref, ns_ref, nd_ref, b_ref, out_ref):
    agg = p_ref[...] * nd_ref[0, :][:, None]
    out_ref[...] = jnp.tanh(agg + b_ref[0, :][None, :]) * ns_ref[0, :][:, None]


_k3 = pl.pallas_call(
    _k3_body,
    grid=(G,),
    in_specs=[
        pl.BlockSpec((BR, DO), lambda i: (i, 0)),
        pl.BlockSpec((1, BR), lambda i: (0, i)),
        pl.BlockSpec((1, BR), lambda i: (0, i)),
        pl.BlockSpec((1, DO), lambda i: (0, 0)),
    ],
    out_specs=pl.BlockSpec((BR, DO), lambda i: (i, 0)),
    out_shape=jax.ShapeDtypeStruct((P, DO), jnp.float32),
)


def _k4_body(p_ref, nd_ref, wmu_ref, bmu_ref, wls_ref, bls_ref, eps_ref,
             mu_ref, sg_ref, z_ref):
    agg = p_ref[...] * nd_ref[0, :][:, None]
    mu = jnp.tanh(jnp.dot(agg, wmu_ref[...],
                          preferred_element_type=jnp.float32) + bmu_ref[0, :][None, :])
    ls = jnp.tanh(jnp.dot(agg, wls_ref[...],
                          preferred_element_type=jnp.float32) + bls_ref[0, :][None, :])
    sg = jnp.exp(ls)
    mu_ref[...] = mu
    sg_ref[...] = sg
    z_ref[...] = mu + sg * eps_ref[...]


_k4 = pl.pallas_call(
    _k4_body,
    grid=(G,),
    in_specs=[
        pl.BlockSpec((BR, DO), lambda i: (i, 0)),
        pl.BlockSpec((1, BR), lambda i: (0, i)),
        pl.BlockSpec((DO, DO), lambda i: (0, 0)),
        pl.BlockSpec((1, DO), lambda i: (0, 0)),
        pl.BlockSpec((DO, DO), lambda i: (0, 0)),
        pl.BlockSpec((1, DO), lambda i: (0, 0)),
        pl.BlockSpec((BR, DO), lambda i: (i, 0)),
    ],
    out_specs=[
        pl.BlockSpec((BR, DO), lambda i: (i, 0)),
        pl.BlockSpec((BR, DO), lambda i: (i, 0)),
        pl.BlockSpec((BR, DO), lambda i: (i, 0)),
    ],
    out_shape=[
        jax.ShapeDtypeStruct((P, DO), jnp.float32),
        jax.ShapeDtypeStruct((P, DO), jnp.float32),
        jax.ShapeDtypeStruct((P, DO), jnp.float32),
    ],
)


def kernel(features, edge_index, W1, b1, W2, b2, Wmu, bmu, Wls, bls, eps):
    src = edge_index[0]
    dst = edge_index[1]
    pad_src = jnp.full((EP - E,), SENT, jnp.int32)
    pad_dst = SENT + (jnp.arange(EP - E, dtype=jnp.int32) % (P - N))
    src2d = jnp.concatenate([src, pad_src]).reshape(EP // CHUNK, CHUNK)
    dst2d = jnp.concatenate([dst, pad_dst]).reshape(EP // CHUNK, CHUNK)

    x = jnp.zeros((P, DI), jnp.float32).at[:N].set(features)
    epsp = jnp.zeros((P, DO), jnp.float32).at[:N].set(eps)

    cnt = _deg(src2d, dst2d)
    cs = cnt[0].reshape(NTILES, 128 * 128)[:, :P]
    cd = cnt[1].reshape(NTILES, 128 * 128)[:, :P]

    h0, ns, nd = _k1(x, W1, cs, cd)
    p1 = _prop(h0, src2d, dst2d)
    h1 = _k2(p1, ns, nd, W2, b1.reshape(1, DO))
    p2 = _prop(h1, src2d, dst2d)
    h2 = _k3(p2, ns, nd, b2.reshape(1, DO))
    p3 = _prop(h2, src2d, dst2d)
    mu, sg, z = _k4(p3, nd, Wmu, bmu.reshape(1, DO), Wls, bls.reshape(1, DO),
                    epsp)
    return mu[:N], sg[:N], z[:N]


# speedup vs baseline: 2.1348x; 2.1348x over previous
import functools

import jax
import jax.numpy as jnp
from jax import lax
from jax.experimental import pallas as pl
from jax.experimental.pallas import tpu as pltpu
from jax.experimental.pallas import tpu_sc as plsc

N = 10000
P = 10240
DI = 128
DO = 64
E = 320000
NC = 2
NS = 16
NTILES = NC * NS
CHUNK = 128
CPT0 = 128
CPT1 = 32
EP = NS * (CPT0 + CPT1) * CHUNK
ROWS0 = NS * CPT0
SENT = N
CNT_ROWS = 256
RPT = P // NS
ZROWS = 160
NBUF = 4

_sc_mesh = plsc.VectorSubcoreMesh(
    core_axis_name="c", subcore_axis_name="s", num_cores=NC, num_subcores=NS
)


def _deg_body(src_hbm, dst_hbm, out_hbm, idx_s, idx_d, cnt_s, cnt_d):
    c = lax.axis_index("c")
    s = lax.axis_index("s")
    tile = c * NS + s
    nch = jnp.where(c == 0, CPT0, CPT1)
    base = jnp.where(c == 0, s * CPT0, ROWS0 + s * CPT1)
    zero16 = jnp.zeros((16,), jnp.float32)
    ones16 = jnp.ones((16,), jnp.float32)

    def zloop(r, carry):
        for cc in range(8):
            cnt_s[r, pl.ds(cc * 16, 16)] = zero16
            cnt_d[r, pl.ds(cc * 16, 16)] = zero16
        return carry

    lax.fori_loop(0, 128, zloop, 0)

    def lloop(i, carry):
        pltpu.sync_copy(src_hbm.at[pl.ds(base + i * 32, 32)],
                        idx_s.at[pl.ds(i * 32, 32)])
        pltpu.sync_copy(dst_hbm.at[pl.ds(base + i * 32, 32)],
                        idx_d.at[pl.ds(i * 32, 32)])
        return carry

    lax.fori_loop(0, nch // 32, lloop, 0)

    def cloop(r, carry):
        for cc in range(8):
            v = idx_s[r, pl.ds(cc * 16, 16)]
            plsc.addupdate_scatter(cnt_s, [v >> 7, v & 127], ones16)
            w = idx_d[r, pl.ds(cc * 16, 16)]
            plsc.addupdate_scatter(cnt_d, [w >> 7, w & 127], ones16)
        return carry

    lax.fori_loop(0, nch, cloop, 0)

    pltpu.sync_copy(cnt_s, out_hbm.at[0, tile])
    pltpu.sync_copy(cnt_d, out_hbm.at[1, tile])


_deg = pl.kernel(
    _deg_body,
    out_type=jax.ShapeDtypeStruct((2, NTILES, 128, 128), jnp.float32),
    mesh=_sc_mesh,
    scratch_types=[
        pltpu.VMEM((CPT0, CHUNK), jnp.int32),
        pltpu.VMEM((CPT0, CHUNK), jnp.int32),
        pltpu.VMEM((128, 128), jnp.float32),
        pltpu.VMEM((128, 128), jnp.float32),
    ],
    compiler_params=pltpu.CompilerParams(needs_layout_passes=False),
)


CPTT = EP // CHUNK // NS
PH = 2
CPP = CPTT // PH


def _prop_body(h_hbm, src_hbm, dst_hbm, out_hbm, idx_s, idx_d, rows, zbuf,
               acc, gsem, ssem):
    c = lax.axis_index("c")
    s = lax.axis_index("s")

    @pl.when(c == 0)
    def _():
        zero16 = jnp.zeros((16,), jnp.float32)

        def zloop(r, carry):
            for cc in range(4):
                zbuf[r, pl.ds(cc * 16, 16)] = zero16
            return carry

        lax.fori_loop(0, ZROWS, zloop, 0)
        for k in range(RPT // ZROWS):
            pltpu.sync_copy(zbuf, acc.at[pl.ds(s * RPT + k * ZROWS, ZROWS)])

        plsc.subcore_barrier()

        def gloop(g, carry):
            g0 = g * NBUF
            cps = [
                pltpu.async_copy(h_hbm.at[idx_s.at[g0 + b]], rows.at[b], gsem)
                for b in range(NBUF)
            ]
            for cp in cps:
                cp.wait()
            cps = [
                pltpu.async_copy(rows.at[b], acc.at[idx_d.at[g0 + b]], ssem,
                                 add=True)
                for b in range(NBUF)
            ]
            for cp in cps:
                cp.wait()
            return carry

        for ph in range(PH):
            pltpu.sync_copy(src_hbm.at[pl.ds(s * CPTT + ph * CPP, CPP)], idx_s)
            pltpu.sync_copy(dst_hbm.at[pl.ds(s * CPTT + ph * CPP, CPP)], idx_d)
            lax.fori_loop(0, CPP // NBUF, gloop, 0)

        plsc.subcore_barrier()
        pltpu.sync_copy(acc.at[pl.ds(s * RPT, RPT)],
                        out_hbm.at[pl.ds(s * RPT, RPT)])


_prop = pl.kernel(
    _prop_body,
    out_type=jax.ShapeDtypeStruct((P, DO), jnp.float32),
    mesh=_sc_mesh,
    scratch_types=[
        pltpu.VMEM((CPP, CHUNK), jnp.int32),
        pltpu.VMEM((CPP, CHUNK), jnp.int32),
        pltpu.VMEM((NBUF, CHUNK, DO), jnp.float32),
        pltpu.VMEM((ZROWS, DO), jnp.float32),
        pltpu.VMEM_SHARED((P, DO), jnp.float32),
        pltpu.SemaphoreType.DMA,
        pltpu.SemaphoreType.DMA,
    ],
    compiler_params=pltpu.CompilerParams(
        needs_layout_passes=False, use_tc_tiling_on_sc=False
    ),
)

BR = 1280
G = P // BR


def _k1_body(x_ref, w_ref, cs_ref, cd_ref, h0_ref, ns_ref, nd_ref):
    cs = jnp.sum(cs_ref[...], axis=0)
    cd = jnp.sum(cd_ref[...], axis=0)
    ns = lax.rsqrt(jnp.maximum(cs, 1.0))
    nd = lax.rsqrt(jnp.maximum(cd, 1.0))
    ns_ref[0, :] = ns
    nd_ref[0, :] = nd
    h0_ref[...] = jnp.dot(x_ref[...], w_ref[...],
                          preferred_element_type=jnp.float32) * ns[:, None]


_k1 = pl.pallas_call(
    _k1_body,
    grid=(G,),
    in_specs=[
        pl.BlockSpec((BR, DI), lambda i: (i, 0)),
        pl.BlockSpec((DI, DO), lambda i: (0, 0)),
        pl.BlockSpec((NTILES, BR), lambda i: (0, i)),
        pl.BlockSpec((NTILES, BR), lambda i: (0, i)),
    ],
    out_specs=[
        pl.BlockSpec((BR, DO), lambda i: (i, 0)),
        pl.BlockSpec((1, BR), lambda i: (0, i)),
        pl.BlockSpec((1, BR), lambda i: (0, i)),
    ],
    out_shape=[
        jax.ShapeDtypeStruct((P, DO), jnp.float32),
        jax.ShapeDtypeStruct((1, P), jnp.float32),
        jax.ShapeDtypeStruct((1, P), jnp.float32),
    ],
)


def _row_mask(shape):
    row0 = pl.program_id(0) * BR
    rid = lax.broadcasted_iota(jnp.int32, shape, 0) + row0
    return jnp.where(rid < N, 1.0, 0.0).astype(jnp.float32)


def _k2_body(p_ref, ns_ref, nd_ref, w_ref, b_ref, out_ref):
    agg = p_ref[...] * nd_ref[0, :][:, None]
    h = jnp.tanh(agg + b_ref[0, :][None, :])
    out_ref[...] = (jnp.dot(h, w_ref[...], preferred_element_type=jnp.float32)
                    * ns_ref[0, :][:, None] * _row_mask((BR, 1)))


_k2 = pl.pallas_call(
    _k2_body,
    grid=(G,),
    in_specs=[
        pl.BlockSpec((BR, DO), lambda i: (i, 0)),
        pl.BlockSpec((1, BR), lambda i: (0, i)),
        pl.BlockSpec((1, BR), lambda i: (0, i)),
        pl.BlockSpec((DO, DO), lambda i: (0, 0)),
        pl.BlockSpec((1, DO), lambda i: (0, 0)),
    ],
    out_specs=pl.BlockSpec((BR, DO), lambda i: (i, 0)),
    out_shape=jax.ShapeDtypeStruct((P, DO), jnp.float32),
)


def _k3_body(p_ref, ns_ref, nd_ref, b_ref, out_ref):
    agg = p_ref[...] * nd_ref[0, :][:, None]
    out_ref[...] = (jnp.tanh(agg + b_ref[0, :][None, :])
                    * ns_ref[0, :][:, None] * _row_mask((BR, 1)))


_k3 = pl.pallas_call(
    _k3_body,
    grid=(G,),
    in_specs=[
        pl.BlockSpec((BR, DO), lambda i: (i, 0)),
        pl.BlockSpec((1, BR), lambda i: (0, i)),
        pl.BlockSpec((1, BR), lambda i: (0, i)),
        pl.BlockSpec((1, DO), lambda i: (0, 0)),
    ],
    out_specs=pl.BlockSpec((BR, DO), lambda i: (i, 0)),
    out_shape=jax.ShapeDtypeStruct((P, DO), jnp.float32),
)


def _k4_body(p_ref, nd_ref, wmu_ref, bmu_ref, wls_ref, bls_ref, eps_ref,
             mu_ref, sg_ref, z_ref):
    agg = p_ref[...] * nd_ref[0, :][:, None]
    mu = jnp.tanh(jnp.dot(agg, wmu_ref[...],
                          preferred_element_type=jnp.float32) + bmu_ref[0, :][None, :])
    ls = jnp.tanh(jnp.dot(agg, wls_ref[...],
                          preferred_element_type=jnp.float32) + bls_ref[0, :][None, :])
    sg = jnp.exp(ls)
    mu_ref[...] = mu
    sg_ref[...] = sg
    z_ref[...] = mu + sg * eps_ref[...]


_k4 = pl.pallas_call(
    _k4_body,
    grid=(G,),
    in_specs=[
        pl.BlockSpec((BR, DO), lambda i: (i, 0)),
        pl.BlockSpec((1, BR), lambda i: (0, i)),
        pl.BlockSpec((DO, DO), lambda i: (0, 0)),
        pl.BlockSpec((1, DO), lambda i: (0, 0)),
        pl.BlockSpec((DO, DO), lambda i: (0, 0)),
        pl.BlockSpec((1, DO), lambda i: (0, 0)),
        pl.BlockSpec((BR, DO), lambda i: (i, 0)),
    ],
    out_specs=[
        pl.BlockSpec((BR, DO), lambda i: (i, 0)),
        pl.BlockSpec((BR, DO), lambda i: (i, 0)),
        pl.BlockSpec((BR, DO), lambda i: (i, 0)),
    ],
    out_shape=[
        jax.ShapeDtypeStruct((P, DO), jnp.float32),
        jax.ShapeDtypeStruct((P, DO), jnp.float32),
        jax.ShapeDtypeStruct((P, DO), jnp.float32),
    ],
)


def kernel(features, edge_index, W1, b1, W2, b2, Wmu, bmu, Wls, bls, eps):
    src = edge_index[0]
    dst = edge_index[1]
    ar = jnp.arange(EP - E, dtype=jnp.int32)
    pad_zero_rows = SENT + ar % (P - N)
    src2d = jnp.concatenate([src, pad_zero_rows]).reshape(EP // CHUNK, CHUNK)
    dstdeg2d = jnp.concatenate([dst, pad_zero_rows]).reshape(EP // CHUNK, CHUNK)
    dst2d = jnp.concatenate([dst, ar % N]).reshape(EP // CHUNK, CHUNK)

    x = jnp.zeros((P, DI), jnp.float32).at[:N].set(features)
    epsp = jnp.zeros((P, DO), jnp.float32).at[:N].set(eps)

    cnt = _deg(src2d, dstdeg2d)
    cs = cnt[0].reshape(NTILES, 128 * 128)[:, :P]
    cd = cnt[1].reshape(NTILES, 128 * 128)[:, :P]

    h0, ns, nd = _k1(x, W1, cs, cd)
    p1 = _prop(h0, src2d, dst2d)
    h1 = _k2(p1, ns, nd, W2, b1.reshape(1, DO))
    p2 = _prop(h1, src2d, dst2d)
    h2 = _k3(p2, ns, nd, b2.reshape(1, DO))
    p3 = _prop(h2, src2d, dst2d)
    mu, sg, z = _k4(p3, nd, Wmu, bmu.reshape(1, DO), Wls, bls.reshape(1, DO),
                    epsp)
    return mu[:N], sg[:N], z[:N]


# speedup vs baseline: 3.1147x; 1.4590x over previous
import functools

import jax
import jax.numpy as jnp
from jax import lax
from jax.experimental import pallas as pl
from jax.experimental.pallas import tpu as pltpu
from jax.experimental.pallas import tpu_sc as plsc

N = 10000
P = 10240
DI = 128
DO = 64
E = 320000
NC = 2
NS = 16
NTILES = NC * NS
CHUNK = 128
CPT = 80
EP = NTILES * CPT * CHUNK
SENT = N
CNT_ROWS = 256
RPT = P // NS
ZROWS = 160
NBUF = 4

_sc_mesh = plsc.VectorSubcoreMesh(
    core_axis_name="c", subcore_axis_name="s", num_cores=NC, num_subcores=NS
)


def _deg_body(src_hbm, dst_hbm, out_hbm, idx_s, idx_d, cnt_s, cnt_d):
    c = lax.axis_index("c")
    s = lax.axis_index("s")
    tile = c * NS + s
    zero16 = jnp.zeros((16,), jnp.float32)
    ones16 = jnp.ones((16,), jnp.float32)

    def zloop(r, carry):
        for cc in range(8):
            cnt_s[r, pl.ds(cc * 16, 16)] = zero16
            cnt_d[r, pl.ds(cc * 16, 16)] = zero16
        return carry

    lax.fori_loop(0, 128, zloop, 0)

    pltpu.sync_copy(src_hbm.at[pl.ds(tile * CPT, CPT)], idx_s)
    pltpu.sync_copy(dst_hbm.at[pl.ds(tile * CPT, CPT)], idx_d)

    def cloop(r, carry):
        for cc in range(8):
            v = idx_s[r, pl.ds(cc * 16, 16)]
            plsc.addupdate_scatter(cnt_s, [v >> 7, v & 127], ones16)
            w = idx_d[r, pl.ds(cc * 16, 16)]
            plsc.addupdate_scatter(cnt_d, [w >> 7, w & 127], ones16)
        return carry

    lax.fori_loop(0, CPT, cloop, 0)

    pltpu.sync_copy(cnt_s, out_hbm.at[0, tile])
    pltpu.sync_copy(cnt_d, out_hbm.at[1, tile])


_deg = pl.kernel(
    _deg_body,
    out_type=jax.ShapeDtypeStruct((2, NTILES, 128, 128), jnp.float32),
    mesh=_sc_mesh,
    scratch_types=[
        pltpu.VMEM((CPT, CHUNK), jnp.int32),
        pltpu.VMEM((CPT, CHUNK), jnp.int32),
        pltpu.VMEM((128, 128), jnp.float32),
        pltpu.VMEM((128, 128), jnp.float32),
    ],
    compiler_params=pltpu.CompilerParams(needs_layout_passes=False),
)


def _prop_body(h_hbm, src_hbm, dst_hbm, out_hbm, idx_s, idx_d, rows, zbuf,
               acc, gsem, ssem):
    c = lax.axis_index("c")
    s = lax.axis_index("s")
    tile = c * NS + s
    zero16 = jnp.zeros((16,), jnp.float32)

    def zloop(r, carry):
        for cc in range(4):
            zbuf[r, pl.ds(cc * 16, 16)] = zero16
        return carry

    lax.fori_loop(0, ZROWS, zloop, 0)
    for k in range(RPT // ZROWS):
        pltpu.sync_copy(zbuf, acc.at[pl.ds(s * RPT + k * ZROWS, ZROWS)])

    pltpu.sync_copy(src_hbm.at[pl.ds(tile * CPT, CPT)], idx_s)
    pltpu.sync_copy(dst_hbm.at[pl.ds(tile * CPT, CPT)], idx_d)
    plsc.subcore_barrier()

    def gloop(g, carry):
        g0 = g * NBUF
        cps = [
            pltpu.async_copy(h_hbm.at[idx_s.at[g0 + b]], rows.at[b], gsem)
            for b in range(NBUF)
        ]
        for cp in cps:
            cp.wait()
        cps = [
            pltpu.async_copy(rows.at[b], acc.at[idx_d.at[g0 + b]], ssem,
                             add=True)
            for b in range(NBUF)
        ]
        for cp in cps:
            cp.wait()
        return carry

    lax.fori_loop(0, CPT // NBUF, gloop, 0)

    plsc.subcore_barrier()
    pltpu.sync_copy(acc.at[pl.ds(s * RPT, RPT)],
                    out_hbm.at[c, pl.ds(s * RPT, RPT)])


_prop = pl.kernel(
    _prop_body,
    out_type=jax.ShapeDtypeStruct((NC, P, DO), jnp.float32),
    mesh=_sc_mesh,
    scratch_types=[
        pltpu.VMEM((CPT, CHUNK), jnp.int32),
        pltpu.VMEM((CPT, CHUNK), jnp.int32),
        pltpu.VMEM((NBUF, CHUNK, DO), jnp.float32),
        pltpu.VMEM((ZROWS, DO), jnp.float32),
        pltpu.VMEM_SHARED((P, DO), jnp.float32),
        pltpu.SemaphoreType.DMA,
        pltpu.SemaphoreType.DMA,
    ],
    compiler_params=pltpu.CompilerParams(
        needs_layout_passes=False, use_tc_tiling_on_sc=False
    ),
)

BR = 1280
G = P // BR


def _k1_body(x_ref, w_ref, cs_ref, cd_ref, h0_ref, ns_ref, nd_ref):
    cs = jnp.sum(cs_ref[...], axis=0)
    cd = jnp.sum(cd_ref[...], axis=0)
    ns = lax.rsqrt(jnp.maximum(cs, 1.0))
    nd = lax.rsqrt(jnp.maximum(cd, 1.0))
    ns_ref[0, :] = ns
    nd_ref[0, :] = nd
    h0_ref[...] = jnp.dot(x_ref[...], w_ref[...],
                          preferred_element_type=jnp.float32) * ns[:, None]


_k1 = pl.pallas_call(
    _k1_body,
    grid=(G,),
    in_specs=[
        pl.BlockSpec((BR, DI), lambda i: (i, 0)),
        pl.BlockSpec((DI, DO), lambda i: (0, 0)),
        pl.BlockSpec((NTILES, BR), lambda i: (0, i)),
        pl.BlockSpec((NTILES, BR), lambda i: (0, i)),
    ],
    out_specs=[
        pl.BlockSpec((BR, DO), lambda i: (i, 0)),
        pl.BlockSpec((1, BR), lambda i: (0, i)),
        pl.BlockSpec((1, BR), lambda i: (0, i)),
    ],
    out_shape=[
        jax.ShapeDtypeStruct((P, DO), jnp.float32),
        jax.ShapeDtypeStruct((1, P), jnp.float32),
        jax.ShapeDtypeStruct((1, P), jnp.float32),
    ],
)


def _row_mask(shape):
    row0 = pl.program_id(0) * BR
    rid = lax.broadcasted_iota(jnp.int32, shape, 0) + row0
    return jnp.where(rid < N, 1.0, 0.0).astype(jnp.float32)


def _k2_body(p_ref, ns_ref, nd_ref, w_ref, b_ref, out_ref):
    agg = (p_ref[0] + p_ref[1]) * nd_ref[0, :][:, None]
    h = jnp.tanh(agg + b_ref[0, :][None, :])
    out_ref[...] = (jnp.dot(h, w_ref[...], preferred_element_type=jnp.float32)
                    * ns_ref[0, :][:, None] * _row_mask((BR, 1)))


_k2 = pl.pallas_call(
    _k2_body,
    grid=(G,),
    in_specs=[
        pl.BlockSpec((NC, BR, DO), lambda i: (0, i, 0)),
        pl.BlockSpec((1, BR), lambda i: (0, i)),
        pl.BlockSpec((1, BR), lambda i: (0, i)),
        pl.BlockSpec((DO, DO), lambda i: (0, 0)),
        pl.BlockSpec((1, DO), lambda i: (0, 0)),
    ],
    out_specs=pl.BlockSpec((BR, DO), lambda i: (i, 0)),
    out_shape=jax.ShapeDtypeStruct((P, DO), jnp.float32),
)


def _k3_body(p_ref, ns_ref, nd_ref, b_ref, out_ref):
    agg = (p_ref[0] + p_ref[1]) * nd_ref[0, :][:, None]
    out_ref[...] = (jnp.tanh(agg + b_ref[0, :][None, :])
                    * ns_ref[0, :][:, None] * _row_mask((BR, 1)))


_k3 = pl.pallas_call(
    _k3_body,
    grid=(G,),
    in_specs=[
        pl.BlockSpec((NC, BR, DO), lambda i: (0, i, 0)),
        pl.BlockSpec((1, BR), lambda i: (0, i)),
        pl.BlockSpec((1, BR), lambda i: (0, i)),
        pl.BlockSpec((1, DO), lambda i: (0, 0)),
    ],
    out_specs=pl.BlockSpec((BR, DO), lambda i: (i, 0)),
    out_shape=jax.ShapeDtypeStruct((P, DO), jnp.float32),
)


def _k4_body(p_ref, nd_ref, wmu_ref, bmu_ref, wls_ref, bls_ref, eps_ref,
             mu_ref, sg_ref, z_ref):
    agg = (p_ref[0] + p_ref[1]) * nd_ref[0, :][:, None]
    mu = jnp.tanh(jnp.dot(agg, wmu_ref[...],
                          preferred_element_type=jnp.float32) + bmu_ref[0, :][None, :])
    ls = jnp.tanh(jnp.dot(agg, wls_ref[...],
                          preferred_element_type=jnp.float32) + bls_ref[0, :][None, :])
    sg = jnp.exp(ls)
    mu_ref[...] = mu
    sg_ref[...] = sg
    z_ref[...] = mu + sg * eps_ref[...]


_k4 = pl.pallas_call(
    _k4_body,
    grid=(G,),
    in_specs=[
        pl.BlockSpec((NC, BR, DO), lambda i: (0, i, 0)),
        pl.BlockSpec((1, BR), lambda i: (0, i)),
        pl.BlockSpec((DO, DO), lambda i: (0, 0)),
        pl.BlockSpec((1, DO), lambda i: (0, 0)),
        pl.BlockSpec((DO, DO), lambda i: (0, 0)),
        pl.BlockSpec((1, DO), lambda i: (0, 0)),
        pl.BlockSpec((BR, DO), lambda i: (i, 0)),
    ],
    out_specs=[
        pl.BlockSpec((BR, DO), lambda i: (i, 0)),
        pl.BlockSpec((BR, DO), lambda i: (i, 0)),
        pl.BlockSpec((BR, DO), lambda i: (i, 0)),
    ],
    out_shape=[
        jax.ShapeDtypeStruct((P, DO), jnp.float32),
        jax.ShapeDtypeStruct((P, DO), jnp.float32),
        jax.ShapeDtypeStruct((P, DO), jnp.float32),
    ],
)


def kernel(features, edge_index, W1, b1, W2, b2, Wmu, bmu, Wls, bls, eps):
    src = edge_index[0]
    dst = edge_index[1]
    ar = jnp.arange(EP - E, dtype=jnp.int32)
    pad_zero_rows = SENT + ar % (P - N)
    src2d = jnp.concatenate([src, pad_zero_rows]).reshape(EP // CHUNK, CHUNK)
    dstdeg2d = jnp.concatenate([dst, pad_zero_rows]).reshape(EP // CHUNK, CHUNK)
    dst2d = jnp.concatenate([dst, ar % N]).reshape(EP // CHUNK, CHUNK)

    x = jnp.zeros((P, DI), jnp.float32).at[:N].set(features)
    epsp = jnp.zeros((P, DO), jnp.float32).at[:N].set(eps)

    cnt = _deg(src2d, dstdeg2d)
    cs = cnt[0].reshape(NTILES, 128 * 128)[:, :P]
    cd = cnt[1].reshape(NTILES, 128 * 128)[:, :P]

    h0, ns, nd = _k1(x, W1, cs, cd)
    p1 = _prop(h0, src2d, dst2d)
    h1 = _k2(p1, ns, nd, W2, b1.reshape(1, DO))
    p2 = _prop(h1, src2d, dst2d)
    h2 = _k3(p2, ns, nd, b2.reshape(1, DO))
    p3 = _prop(h2, src2d, dst2d)
    mu, sg, z = _k4(p3, nd, Wmu, bmu.reshape(1, DO), Wls, bls.reshape(1, DO),
                    epsp)
    return mu[:N], sg[:N], z[:N]


# speedup vs baseline: 3.1884x; 1.0236x over previous
import functools

import jax
import jax.numpy as jnp
from jax import lax
from jax.experimental import pallas as pl
from jax.experimental.pallas import tpu as pltpu
from jax.experimental.pallas import tpu_sc as plsc

N = 10000
P = 10240
DI = 128
DO = 64
E = 320000
NC = 2
NS = 16
NTILES = NC * NS
CHUNK = 128
CPT = 80
EP = NTILES * CPT * CHUNK
SENT = N
CNT_ROWS = 256
RPT = P // NS
ZROWS = 160
NBUF = 8

_sc_mesh = plsc.VectorSubcoreMesh(
    core_axis_name="c", subcore_axis_name="s", num_cores=NC, num_subcores=NS
)


def _deg_body(src_hbm, dst_hbm, out_hbm, idx_s, idx_d, cnt_s, cnt_d):
    c = lax.axis_index("c")
    s = lax.axis_index("s")
    tile = c * NS + s
    zero16 = jnp.zeros((16,), jnp.float32)
    ones16 = jnp.ones((16,), jnp.float32)

    def zloop(r, carry):
        for cc in range(8):
            cnt_s[r, pl.ds(cc * 16, 16)] = zero16
            cnt_d[r, pl.ds(cc * 16, 16)] = zero16
        return carry

    lax.fori_loop(0, 128, zloop, 0)

    pltpu.sync_copy(src_hbm.at[pl.ds(tile * CPT, CPT)], idx_s)
    pltpu.sync_copy(dst_hbm.at[pl.ds(tile * CPT, CPT)], idx_d)

    def cloop(r, carry):
        for cc in range(8):
            v = idx_s[r, pl.ds(cc * 16, 16)]
            plsc.addupdate_scatter(cnt_s, [v >> 7, v & 127], ones16)
            w = idx_d[r, pl.ds(cc * 16, 16)]
            plsc.addupdate_scatter(cnt_d, [w >> 7, w & 127], ones16)
        return carry

    lax.fori_loop(0, CPT, cloop, 0)

    pltpu.sync_copy(cnt_s, out_hbm.at[0, tile])
    pltpu.sync_copy(cnt_d, out_hbm.at[1, tile])


_deg = pl.kernel(
    _deg_body,
    out_type=jax.ShapeDtypeStruct((2, NTILES, 128, 128), jnp.float32),
    mesh=_sc_mesh,
    scratch_types=[
        pltpu.VMEM((CPT, CHUNK), jnp.int32),
        pltpu.VMEM((CPT, CHUNK), jnp.int32),
        pltpu.VMEM((128, 128), jnp.float32),
        pltpu.VMEM((128, 128), jnp.float32),
    ],
    compiler_params=pltpu.CompilerParams(needs_layout_passes=False),
)


def _prop_body(h_hbm, src_hbm, dst_hbm, out_hbm, idx_s, idx_d, rows, zbuf,
               acc, gsem, ssem):
    c = lax.axis_index("c")
    s = lax.axis_index("s")
    tile = c * NS + s
    zero16 = jnp.zeros((16,), jnp.float32)

    def zloop(r, carry):
        for cc in range(4):
            zbuf[r, pl.ds(cc * 16, 16)] = zero16
        return carry

    lax.fori_loop(0, ZROWS, zloop, 0)
    for k in range(RPT // ZROWS):
        pltpu.sync_copy(zbuf, acc.at[pl.ds(s * RPT + k * ZROWS, ZROWS)])

    plsc.subcore_barrier()

    def gloop(g, carry):
        g0 = g * NBUF
        cps = [
            pltpu.async_copy(h_hbm.at[idx_s.at[g0 + b]], rows.at[b], gsem)
            for b in range(NBUF)
        ]
        for cp in cps:
            cp.wait()
        cps = [
            pltpu.async_copy(rows.at[b], acc.at[idx_d.at[g0 + b]], ssem,
                             add=True)
            for b in range(NBUF)
        ]
        for cp in cps:
            cp.wait()
        return carry

    for ph in range(2):
        pltpu.sync_copy(src_hbm.at[pl.ds(tile * CPT + ph * (CPT // 2), CPT // 2)],
                        idx_s)
        pltpu.sync_copy(dst_hbm.at[pl.ds(tile * CPT + ph * (CPT // 2), CPT // 2)],
                        idx_d)
        lax.fori_loop(0, CPT // 2 // NBUF, gloop, 0)

    plsc.subcore_barrier()
    pltpu.sync_copy(acc.at[pl.ds(s * RPT, RPT)],
                    out_hbm.at[c, pl.ds(s * RPT, RPT)])


_prop = pl.kernel(
    _prop_body,
    out_type=jax.ShapeDtypeStruct((NC, P, DO), jnp.float32),
    mesh=_sc_mesh,
    scratch_types=[
        pltpu.VMEM((CPT // 2, CHUNK), jnp.int32),
        pltpu.VMEM((CPT // 2, CHUNK), jnp.int32),
        pltpu.VMEM((NBUF, CHUNK, DO), jnp.float32),
        pltpu.VMEM((ZROWS, DO), jnp.float32),
        pltpu.VMEM_SHARED((P, DO), jnp.float32),
        pltpu.SemaphoreType.DMA,
        pltpu.SemaphoreType.DMA,
    ],
    compiler_params=pltpu.CompilerParams(
        needs_layout_passes=False, use_tc_tiling_on_sc=False
    ),
)

BR = 1280
G = P // BR


def _k1_body(x_ref, w_ref, cs_ref, cd_ref, h0_ref, ns_ref, nd_ref):
    cs = jnp.sum(cs_ref[...], axis=0)
    cd = jnp.sum(cd_ref[...], axis=0)
    ns = lax.rsqrt(jnp.maximum(cs, 1.0))
    nd = lax.rsqrt(jnp.maximum(cd, 1.0))
    ns_ref[0, :] = ns
    nd_ref[0, :] = nd
    h0_ref[...] = jnp.dot(x_ref[...], w_ref[...],
                          preferred_element_type=jnp.float32) * ns[:, None]


_k1 = pl.pallas_call(
    _k1_body,
    grid=(G,),
    in_specs=[
        pl.BlockSpec((BR, DI), lambda i: (i, 0)),
        pl.BlockSpec((DI, DO), lambda i: (0, 0)),
        pl.BlockSpec((NTILES, BR), lambda i: (0, i)),
        pl.BlockSpec((NTILES, BR), lambda i: (0, i)),
    ],
    out_specs=[
        pl.BlockSpec((BR, DO), lambda i: (i, 0)),
        pl.BlockSpec((1, BR), lambda i: (0, i)),
        pl.BlockSpec((1, BR), lambda i: (0, i)),
    ],
    out_shape=[
        jax.ShapeDtypeStruct((P, DO), jnp.float32),
        jax.ShapeDtypeStruct((1, P), jnp.float32),
        jax.ShapeDtypeStruct((1, P), jnp.float32),
    ],
)


def _row_mask(shape):
    row0 = pl.program_id(0) * BR
    rid = lax.broadcasted_iota(jnp.int32, shape, 0) + row0
    return jnp.where(rid < N, 1.0, 0.0).astype(jnp.float32)


def _k2_body(p_ref, ns_ref, nd_ref, w_ref, b_ref, out_ref):
    agg = (p_ref[0] + p_ref[1]) * nd_ref[0, :][:, None]
    h = jnp.tanh(agg + b_ref[0, :][None, :])
    out_ref[...] = (jnp.dot(h, w_ref[...], preferred_element_type=jnp.float32)
                    * ns_ref[0, :][:, None] * _row_mask((BR, 1)))


_k2 = pl.pallas_call(
    _k2_body,
    grid=(G,),
    in_specs=[
        pl.BlockSpec((NC, BR, DO), lambda i: (0, i, 0)),
        pl.BlockSpec((1, BR), lambda i: (0, i)),
        pl.BlockSpec((1, BR), lambda i: (0, i)),
        pl.BlockSpec((DO, DO), lambda i: (0, 0)),
        pl.BlockSpec((1, DO), lambda i: (0, 0)),
    ],
    out_specs=pl.BlockSpec((BR, DO), lambda i: (i, 0)),
    out_shape=jax.ShapeDtypeStruct((P, DO), jnp.float32),
)


def _k3_body(p_ref, ns_ref, nd_ref, b_ref, out_ref):
    agg = (p_ref[0] + p_ref[1]) * nd_ref[0, :][:, None]
    out_ref[...] = (jnp.tanh(agg + b_ref[0, :][None, :])
                    * ns_ref[0, :][:, None] * _row_mask((BR, 1)))


_k3 = pl.pallas_call(
    _k3_body,
    grid=(G,),
    in_specs=[
        pl.BlockSpec((NC, BR, DO), lambda i: (0, i, 0)),
        pl.BlockSpec((1, BR), lambda i: (0, i)),
        pl.BlockSpec((1, BR), lambda i: (0, i)),
        pl.BlockSpec((1, DO), lambda i: (0, 0)),
    ],
    out_specs=pl.BlockSpec((BR, DO), lambda i: (i, 0)),
    out_shape=jax.ShapeDtypeStruct((P, DO), jnp.float32),
)


def _k4_body(p_ref, nd_ref, wmu_ref, bmu_ref, wls_ref, bls_ref, eps_ref,
             mu_ref, sg_ref, z_ref):
    agg = (p_ref[0] + p_ref[1]) * nd_ref[0, :][:, None]
    mu = jnp.tanh(jnp.dot(agg, wmu_ref[...],
                          preferred_element_type=jnp.float32) + bmu_ref[0, :][None, :])
    ls = jnp.tanh(jnp.dot(agg, wls_ref[...],
                          preferred_element_type=jnp.float32) + bls_ref[0, :][None, :])
    sg = jnp.exp(ls)
    mu_ref[...] = mu
    sg_ref[...] = sg
    z_ref[...] = mu + sg * eps_ref[...]


_k4 = pl.pallas_call(
    _k4_body,
    grid=(G,),
    in_specs=[
        pl.BlockSpec((NC, BR, DO), lambda i: (0, i, 0)),
        pl.BlockSpec((1, BR), lambda i: (0, i)),
        pl.BlockSpec((DO, DO), lambda i: (0, 0)),
        pl.BlockSpec((1, DO), lambda i: (0, 0)),
        pl.BlockSpec((DO, DO), lambda i: (0, 0)),
        pl.BlockSpec((1, DO), lambda i: (0, 0)),
        pl.BlockSpec((BR, DO), lambda i: (i, 0)),
    ],
    out_specs=[
        pl.BlockSpec((BR, DO), lambda i: (i, 0)),
        pl.BlockSpec((BR, DO), lambda i: (i, 0)),
        pl.BlockSpec((BR, DO), lambda i: (i, 0)),
    ],
    out_shape=[
        jax.ShapeDtypeStruct((P, DO), jnp.float32),
        jax.ShapeDtypeStruct((P, DO), jnp.float32),
        jax.ShapeDtypeStruct((P, DO), jnp.float32),
    ],
)


def kernel(features, edge_index, W1, b1, W2, b2, Wmu, bmu, Wls, bls, eps):
    src = edge_index[0]
    dst = edge_index[1]
    ar = jnp.arange(EP - E, dtype=jnp.int32)
    pad_zero_rows = SENT + ar % (P - N)
    src2d = jnp.concatenate([src, pad_zero_rows]).reshape(EP // CHUNK, CHUNK)
    dstdeg2d = jnp.concatenate([dst, pad_zero_rows]).reshape(EP // CHUNK, CHUNK)
    dst2d = jnp.concatenate([dst, ar % N]).reshape(EP // CHUNK, CHUNK)

    x = jnp.zeros((P, DI), jnp.float32).at[:N].set(features)
    epsp = jnp.zeros((P, DO), jnp.float32).at[:N].set(eps)

    cnt = _deg(src2d, dstdeg2d)
    cs = cnt[0].reshape(NTILES, 128 * 128)[:, :P]
    cd = cnt[1].reshape(NTILES, 128 * 128)[:, :P]

    h0, ns, nd = _k1(x, W1, cs, cd)
    p1 = _prop(h0, src2d, dst2d)
    h1 = _k2(p1, ns, nd, W2, b1.reshape(1, DO))
    p2 = _prop(h1, src2d, dst2d)
    h2 = _k3(p2, ns, nd, b2.reshape(1, DO))
    p3 = _prop(h2, src2d, dst2d)
    mu, sg, z = _k4(p3, nd, Wmu, bmu.reshape(1, DO), Wls, bls.reshape(1, DO),
                    epsp)
    return mu[:N], sg[:N], z[:N]


# speedup vs baseline: 3.2382x; 1.0156x over previous
import functools

import jax
import jax.numpy as jnp
from jax import lax
from jax.experimental import pallas as pl
from jax.experimental.pallas import tpu as pltpu
from jax.experimental.pallas import tpu_sc as plsc

N = 10000
P = 10240
DI = 128
DO = 64
E = 320000
NC = 2
NS = 16
NTILES = NC * NS
CHUNK = 128
CPT = 80
EP = NTILES * CPT * CHUNK
SENT = N
CNT_ROWS = 256
RPT = P // NS
ZROWS = 160
NBUF = 8

_sc_mesh = plsc.VectorSubcoreMesh(
    core_axis_name="c", subcore_axis_name="s", num_cores=NC, num_subcores=NS
)


def _deg_body(src_hbm, dst_hbm, out_hbm, idx_s, idx_d, cnt_s, cnt_d):
    c = lax.axis_index("c")
    s = lax.axis_index("s")
    tile = c * NS + s
    zero16 = jnp.zeros((16,), jnp.float32)
    ones16 = jnp.ones((16,), jnp.float32)

    def zloop(r, carry):
        for cc in range(8):
            cnt_s[r, pl.ds(cc * 16, 16)] = zero16
            cnt_d[r, pl.ds(cc * 16, 16)] = zero16
        return carry

    lax.fori_loop(0, 128, zloop, 0)

    pltpu.sync_copy(src_hbm.at[pl.ds(tile * CPT, CPT)], idx_s)
    pltpu.sync_copy(dst_hbm.at[pl.ds(tile * CPT, CPT)], idx_d)

    def cloop(r, carry):
        for cc in range(8):
            v = idx_s[r, pl.ds(cc * 16, 16)]
            plsc.addupdate_scatter(cnt_s, [v >> 7, v & 127], ones16)
            w = idx_d[r, pl.ds(cc * 16, 16)]
            plsc.addupdate_scatter(cnt_d, [w >> 7, w & 127], ones16)
        return carry

    lax.fori_loop(0, CPT, cloop, 0)

    pltpu.sync_copy(cnt_s, out_hbm.at[0, tile])
    pltpu.sync_copy(cnt_d, out_hbm.at[1, tile])


_deg = pl.kernel(
    _deg_body,
    out_type=jax.ShapeDtypeStruct((2, NTILES, 128, 128), jnp.float32),
    mesh=_sc_mesh,
    scratch_types=[
        pltpu.VMEM((CPT, CHUNK), jnp.int32),
        pltpu.VMEM((CPT, CHUNK), jnp.int32),
        pltpu.VMEM((128, 128), jnp.float32),
        pltpu.VMEM((128, 128), jnp.float32),
    ],
    compiler_params=pltpu.CompilerParams(needs_layout_passes=False),
)


def _prop_body(h_hbm, src_hbm, dst_hbm, out_hbm, idx_s, idx_d, rows, zbuf,
               acc, gsem, ssem):
    c = lax.axis_index("c")
    s = lax.axis_index("s")
    tile = c * NS + s
    zero16 = jnp.zeros((16,), jnp.float32)

    def zloop(r, carry):
        for cc in range(4):
            zbuf[r, pl.ds(cc * 16, 16)] = zero16
        return carry

    lax.fori_loop(0, ZROWS, zloop, 0)
    for k in range(RPT // ZROWS):
        pltpu.sync_copy(zbuf, acc.at[pl.ds(s * RPT + k * ZROWS, ZROWS)])

    plsc.subcore_barrier()

    def gloop(g, carry):
        g0 = g * NBUF
        cps = [
            pltpu.async_copy(h_hbm.at[idx_s.at[g0 + b]], rows.at[b], gsem)
            for b in range(NBUF)
        ]
        for cp in cps:
            cp.wait()
        cps = [
            pltpu.async_copy(rows.at[b], acc.at[idx_d.at[g0 + b]], ssem,
                             add=True)
            for b in range(NBUF)
        ]
        for cp in cps:
            cp.wait()
        return carry

    for ph in range(2):
        pltpu.sync_copy(src_hbm.at[pl.ds(tile * CPT + ph * (CPT // 2), CPT // 2)],
                        idx_s)
        pltpu.sync_copy(dst_hbm.at[pl.ds(tile * CPT + ph * (CPT // 2), CPT // 2)],
                        idx_d)
        lax.fori_loop(0, CPT // 2 // NBUF, gloop, 0)

    plsc.subcore_barrier()
    pltpu.sync_copy(acc.at[pl.ds(s * RPT, RPT)],
                    out_hbm.at[c, pl.ds(s * RPT, RPT)])


_prop = pl.kernel(
    _prop_body,
    out_type=jax.ShapeDtypeStruct((NC, P, DO), jnp.float32),
    mesh=_sc_mesh,
    scratch_types=[
        pltpu.VMEM((CPT // 2, CHUNK), jnp.int32),
        pltpu.VMEM((CPT // 2, CHUNK), jnp.int32),
        pltpu.VMEM((NBUF, CHUNK, DO), jnp.float32),
        pltpu.VMEM((ZROWS, DO), jnp.float32),
        pltpu.VMEM_SHARED((P, DO), jnp.float32),
        pltpu.SemaphoreType.DMA,
        pltpu.SemaphoreType.DMA,
    ],
    compiler_params=pltpu.CompilerParams(
        needs_layout_passes=False, use_tc_tiling_on_sc=False
    ),
)

BR = 1280
G = P // BR


BR1 = 1024


def _k1_body(x_ref, w_ref, cs_ref, cd_ref, h0_ref, ns_ref, nd_ref):
    cs = jnp.sum(cs_ref[0].reshape(NTILES, BR1), axis=0)
    cd = jnp.sum(cd_ref[0].reshape(NTILES, BR1), axis=0)
    ns = lax.rsqrt(jnp.maximum(cs, 1.0))
    nd = lax.rsqrt(jnp.maximum(cd, 1.0))
    ns_ref[0, :] = ns
    nd_ref[0, :] = nd
    h0_ref[...] = jnp.dot(x_ref[...], w_ref[...],
                          preferred_element_type=jnp.float32) * ns[:, None]


_k1 = pl.pallas_call(
    _k1_body,
    grid=(P // BR1,),
    in_specs=[
        pl.BlockSpec((BR1, DI), lambda i: (i, 0)),
        pl.BlockSpec((DI, DO), lambda i: (0, 0)),
        pl.BlockSpec((1, NTILES, BR1 // 128, 128), lambda i: (0, 0, i, 0)),
        pl.BlockSpec((1, NTILES, BR1 // 128, 128), lambda i: (1, 0, i, 0)),
    ],
    out_specs=[
        pl.BlockSpec((BR1, DO), lambda i: (i, 0)),
        pl.BlockSpec((1, BR1), lambda i: (0, i)),
        pl.BlockSpec((1, BR1), lambda i: (0, i)),
    ],
    out_shape=[
        jax.ShapeDtypeStruct((P, DO), jnp.float32),
        jax.ShapeDtypeStruct((1, P), jnp.float32),
        jax.ShapeDtypeStruct((1, P), jnp.float32),
    ],
)


def _row_mask(shape):
    row0 = pl.program_id(0) * BR
    rid = lax.broadcasted_iota(jnp.int32, shape, 0) + row0
    return jnp.where(rid < N, 1.0, 0.0).astype(jnp.float32)


def _k2_body(p_ref, ns_ref, nd_ref, w_ref, b_ref, out_ref):
    agg = (p_ref[0] + p_ref[1]) * nd_ref[0, :][:, None]
    h = jnp.tanh(agg + b_ref[0, :][None, :])
    out_ref[...] = (jnp.dot(h, w_ref[...], preferred_element_type=jnp.float32)
                    * ns_ref[0, :][:, None] * _row_mask((BR, 1)))


_k2 = pl.pallas_call(
    _k2_body,
    grid=(G,),
    in_specs=[
        pl.BlockSpec((NC, BR, DO), lambda i: (0, i, 0)),
        pl.BlockSpec((1, BR), lambda i: (0, i)),
        pl.BlockSpec((1, BR), lambda i: (0, i)),
        pl.BlockSpec((DO, DO), lambda i: (0, 0)),
        pl.BlockSpec((1, DO), lambda i: (0, 0)),
    ],
    out_specs=pl.BlockSpec((BR, DO), lambda i: (i, 0)),
    out_shape=jax.ShapeDtypeStruct((P, DO), jnp.float32),
)


def _k3_body(p_ref, ns_ref, nd_ref, b_ref, out_ref):
    agg = (p_ref[0] + p_ref[1]) * nd_ref[0, :][:, None]
    out_ref[...] = (jnp.tanh(agg + b_ref[0, :][None, :])
                    * ns_ref[0, :][:, None] * _row_mask((BR, 1)))


_k3 = pl.pallas_call(
    _k3_body,
    grid=(G,),
    in_specs=[
        pl.BlockSpec((NC, BR, DO), lambda i: (0, i, 0)),
        pl.BlockSpec((1, BR), lambda i: (0, i)),
        pl.BlockSpec((1, BR), lambda i: (0, i)),
        pl.BlockSpec((1, DO), lambda i: (0, 0)),
    ],
    out_specs=pl.BlockSpec((BR, DO), lambda i: (i, 0)),
    out_shape=jax.ShapeDtypeStruct((P, DO), jnp.float32),
)


def _k4_body(p_ref, nd_ref, wmu_ref, bmu_ref, wls_ref, bls_ref, eps_ref,
             mu_ref, sg_ref, z_ref):
    agg = (p_ref[0] + p_ref[1]) * nd_ref[0, :][:, None]
    mu = jnp.tanh(jnp.dot(agg, wmu_ref[...],
                          preferred_element_type=jnp.float32) + bmu_ref[0, :][None, :])
    ls = jnp.tanh(jnp.dot(agg, wls_ref[...],
                          preferred_element_type=jnp.float32) + bls_ref[0, :][None, :])
    sg = jnp.exp(ls)
    mu_ref[...] = mu
    sg_ref[...] = sg
    z_ref[...] = mu + sg * eps_ref[...]


_k4 = pl.pallas_call(
    _k4_body,
    grid=(G,),
    in_specs=[
        pl.BlockSpec((NC, BR, DO), lambda i: (0, i, 0)),
        pl.BlockSpec((1, BR), lambda i: (0, i)),
        pl.BlockSpec((DO, DO), lambda i: (0, 0)),
        pl.BlockSpec((1, DO), lambda i: (0, 0)),
        pl.BlockSpec((DO, DO), lambda i: (0, 0)),
        pl.BlockSpec((1, DO), lambda i: (0, 0)),
        pl.BlockSpec((BR, DO), lambda i: (i, 0)),
    ],
    out_specs=[
        pl.BlockSpec((BR, DO), lambda i: (i, 0)),
        pl.BlockSpec((BR, DO), lambda i: (i, 0)),
        pl.BlockSpec((BR, DO), lambda i: (i, 0)),
    ],
    out_shape=[
        jax.ShapeDtypeStruct((P, DO), jnp.float32),
        jax.ShapeDtypeStruct((P, DO), jnp.float32),
        jax.ShapeDtypeStruct((P, DO), jnp.float32),
    ],
)


def kernel(features, edge_index, W1, b1, W2, b2, Wmu, bmu, Wls, bls, eps):
    src = edge_index[0]
    dst = edge_index[1]
    ar = jnp.arange(EP - E, dtype=jnp.int32)
    pad_zero_rows = SENT + ar % (P - N)
    src2d = jnp.concatenate([src, pad_zero_rows]).reshape(EP // CHUNK, CHUNK)
    dstdeg2d = jnp.concatenate([dst, pad_zero_rows]).reshape(EP // CHUNK, CHUNK)
    dst2d = jnp.concatenate([dst, ar % N]).reshape(EP // CHUNK, CHUNK)

    x = jnp.zeros((P, DI), jnp.float32).at[:N].set(features)
    epsp = jnp.zeros((P, DO), jnp.float32).at[:N].set(eps)

    cnt = _deg(src2d, dstdeg2d)

    h0, ns, nd = _k1(x, W1, cnt, cnt)
    p1 = _prop(h0, src2d, dst2d)
    h1 = _k2(p1, ns, nd, W2, b1.reshape(1, DO))
    p2 = _prop(h1, src2d, dst2d)
    h2 = _k3(p2, ns, nd, b2.reshape(1, DO))
    p3 = _prop(h2, src2d, dst2d)
    mu, sg, z = _k4(p3, nd, Wmu, bmu.reshape(1, DO), Wls, bls.reshape(1, DO),
                    epsp)
    return mu[:N], sg[:N], z[:N]


# speedup vs baseline: 3.2498x; 1.0036x over previous
import functools

import jax
import jax.numpy as jnp
from jax import lax
from jax.experimental import pallas as pl
from jax.experimental.pallas import tpu as pltpu
from jax.experimental.pallas import tpu_sc as plsc

N = 10000
P = 10240
DI = 128
DO = 64
E = 320000
NC = 2
NS = 16
NTILES = NC * NS
CHUNK = 128
CPT = 80
EP = NTILES * CPT * CHUNK
SENT = N
CNT_ROWS = 256
RPT = P // NS
ZROWS = 160
NBUF = 4

_sc_mesh = plsc.VectorSubcoreMesh(
    core_axis_name="c", subcore_axis_name="s", num_cores=NC, num_subcores=NS
)


def _deg_body(src_hbm, dst_hbm, out_hbm, idx_s, idx_d, cnt_s, cnt_d):
    c = lax.axis_index("c")
    s = lax.axis_index("s")
    tile = c * NS + s
    zero16 = jnp.zeros((16,), jnp.float32)
    ones16 = jnp.ones((16,), jnp.float32)

    def zloop(r, carry):
        for cc in range(8):
            cnt_s[r, pl.ds(cc * 16, 16)] = zero16
            cnt_d[r, pl.ds(cc * 16, 16)] = zero16
        return carry

    lax.fori_loop(0, 128, zloop, 0)

    pltpu.sync_copy(src_hbm.at[pl.ds(tile * CPT, CPT)], idx_s)
    pltpu.sync_copy(dst_hbm.at[pl.ds(tile * CPT, CPT)], idx_d)

    def cloop(r, carry):
        for cc in range(8):
            v = idx_s[r, pl.ds(cc * 16, 16)]
            plsc.addupdate_scatter(cnt_s, [v >> 7, v & 127], ones16)
            w = idx_d[r, pl.ds(cc * 16, 16)]
            plsc.addupdate_scatter(cnt_d, [w >> 7, w & 127], ones16)
        return carry

    lax.fori_loop(0, CPT, cloop, 0)

    pltpu.sync_copy(cnt_s, out_hbm.at[0, tile])
    pltpu.sync_copy(cnt_d, out_hbm.at[1, tile])


_deg = pl.kernel(
    _deg_body,
    out_type=jax.ShapeDtypeStruct((2, NTILES, 128, 128), jnp.float32),
    mesh=_sc_mesh,
    scratch_types=[
        pltpu.VMEM((CPT, CHUNK), jnp.int32),
        pltpu.VMEM((CPT, CHUNK), jnp.int32),
        pltpu.VMEM((128, 128), jnp.float32),
        pltpu.VMEM((128, 128), jnp.float32),
    ],
    compiler_params=pltpu.CompilerParams(needs_layout_passes=False),
)


def _prop_body(h_hbm, src_hbm, dst_hbm, out_hbm, idx_s, idx_d, rows, zbuf,
               acc, gsa, gsb, ssa, ssb):
    c = lax.axis_index("c")
    s = lax.axis_index("s")
    tile = c * NS + s
    zero16 = jnp.zeros((16,), jnp.float32)

    def zloop(r, carry):
        for cc in range(4):
            zbuf[r, pl.ds(cc * 16, 16)] = zero16
        return carry

    lax.fori_loop(0, ZROWS, zloop, 0)
    for k in range(RPT // ZROWS):
        pltpu.sync_copy(zbuf, acc.at[pl.ds(s * RPT + k * ZROWS, ZROWS)])

    plsc.subcore_barrier()

    NGP = CPT // 2 // NBUF

    def fire_g(cb, slot0, sem):
        return [pltpu.async_copy(h_hbm.at[idx_s.at[cb + b]],
                                 rows.at[slot0 + b], sem)
                for b in range(NBUF)]

    def fire_s(cb, slot0, sem):
        return [pltpu.async_copy(rows.at[slot0 + b],
                                 acc.at[idx_d.at[cb + b]], sem, add=True)
                for b in range(NBUF)]

    def drain_g(sem):
        for b in range(NBUF):
            pltpu.make_async_copy(h_hbm.at[pl.ds(0, CHUNK)],
                                  rows.at[b], sem).wait()

    def pair(cb, last):
        d_gb = fire_g(cb + NBUF, NBUF, gsb)
        drain_g(gsa)
        d_sa = fire_s(cb, 0, ssa)
        for cp in d_gb:
            cp.wait()
        d_sb = fire_s(cb + NBUF, NBUF, ssb)
        for cp in d_sa:
            cp.wait()
        if not last:
            fire_g(cb + 2 * NBUF, 0, gsa)
        for cp in d_sb:
            cp.wait()

    def pbody(k, carry):
        pair(k * 2 * NBUF, False)
        return carry

    for ph in range(2):
        pltpu.sync_copy(src_hbm.at[pl.ds(tile * CPT + ph * (CPT // 2), CPT // 2)],
                        idx_s)
        pltpu.sync_copy(dst_hbm.at[pl.ds(tile * CPT + ph * (CPT // 2), CPT // 2)],
                        idx_d)
        fire_g(0, 0, gsa)
        lax.fori_loop(0, NGP // 2 - 1, pbody, 0)
        pair((NGP - 2) * NBUF, True)

    plsc.subcore_barrier()
    pltpu.sync_copy(acc.at[pl.ds(s * RPT, RPT)],
                    out_hbm.at[c, pl.ds(s * RPT, RPT)])


_prop = pl.kernel(
    _prop_body,
    out_type=jax.ShapeDtypeStruct((NC, P, DO), jnp.float32),
    mesh=_sc_mesh,
    scratch_types=[
        pltpu.VMEM((CPT // 2, CHUNK), jnp.int32),
        pltpu.VMEM((CPT // 2, CHUNK), jnp.int32),
        pltpu.VMEM((2 * NBUF, CHUNK, DO), jnp.float32),
        pltpu.VMEM((ZROWS, DO), jnp.float32),
        pltpu.VMEM_SHARED((P, DO), jnp.float32),
        pltpu.SemaphoreType.DMA,
        pltpu.SemaphoreType.DMA,
        pltpu.SemaphoreType.DMA,
        pltpu.SemaphoreType.DMA,
    ],
    compiler_params=pltpu.CompilerParams(
        needs_layout_passes=False, use_tc_tiling_on_sc=False
    ),
)

BR = 1280
G = P // BR


BR1 = 1024


def _k1_body(x_ref, w_ref, cs_ref, cd_ref, h0_ref, ns_ref, nd_ref):
    cs = jnp.sum(cs_ref[0].reshape(NTILES, BR1), axis=0)
    cd = jnp.sum(cd_ref[0].reshape(NTILES, BR1), axis=0)
    ns = lax.rsqrt(jnp.maximum(cs, 1.0))
    nd = lax.rsqrt(jnp.maximum(cd, 1.0))
    ns_ref[0, :] = ns
    nd_ref[0, :] = nd
    h0_ref[...] = jnp.dot(x_ref[...], w_ref[...],
                          preferred_element_type=jnp.float32) * ns[:, None]


_k1 = pl.pallas_call(
    _k1_body,
    grid=(P // BR1,),
    in_specs=[
        pl.BlockSpec((BR1, DI), lambda i: (i, 0)),
        pl.BlockSpec((DI, DO), lambda i: (0, 0)),
        pl.BlockSpec((1, NTILES, BR1 // 128, 128), lambda i: (0, 0, i, 0)),
        pl.BlockSpec((1, NTILES, BR1 // 128, 128), lambda i: (1, 0, i, 0)),
    ],
    out_specs=[
        pl.BlockSpec((BR1, DO), lambda i: (i, 0)),
        pl.BlockSpec((1, BR1), lambda i: (0, i)),
        pl.BlockSpec((1, BR1), lambda i: (0, i)),
    ],
    out_shape=[
        jax.ShapeDtypeStruct((P, DO), jnp.float32),
        jax.ShapeDtypeStruct((1, P), jnp.float32),
        jax.ShapeDtypeStruct((1, P), jnp.float32),
    ],
)


def _row_mask(shape):
    row0 = pl.program_id(0) * BR
    rid = lax.broadcasted_iota(jnp.int32, shape, 0) + row0
    return jnp.where(rid < N, 1.0, 0.0).astype(jnp.float32)


def _k2_body(p_ref, ns_ref, nd_ref, w_ref, b_ref, out_ref):
    agg = (p_ref[0] + p_ref[1]) * nd_ref[0, :][:, None]
    h = jnp.tanh(agg + b_ref[0, :][None, :])
    out_ref[...] = (jnp.dot(h, w_ref[...], preferred_element_type=jnp.float32)
                    * ns_ref[0, :][:, None] * _row_mask((BR, 1)))


_k2 = pl.pallas_call(
    _k2_body,
    grid=(G,),
    in_specs=[
        pl.BlockSpec((NC, BR, DO), lambda i: (0, i, 0)),
        pl.BlockSpec((1, BR), lambda i: (0, i)),
        pl.BlockSpec((1, BR), lambda i: (0, i)),
        pl.BlockSpec((DO, DO), lambda i: (0, 0)),
        pl.BlockSpec((1, DO), lambda i: (0, 0)),
    ],
    out_specs=pl.BlockSpec((BR, DO), lambda i: (i, 0)),
    out_shape=jax.ShapeDtypeStruct((P, DO), jnp.float32),
)


def _k3_body(p_ref, ns_ref, nd_ref, b_ref, out_ref):
    agg = (p_ref[0] + p_ref[1]) * nd_ref[0, :][:, None]
    out_ref[...] = (jnp.tanh(agg + b_ref[0, :][None, :])
                    * ns_ref[0, :][:, None] * _row_mask((BR, 1)))


_k3 = pl.pallas_call(
    _k3_body,
    grid=(G,),
    in_specs=[
        pl.BlockSpec((NC, BR, DO), lambda i: (0, i, 0)),
        pl.BlockSpec((1, BR), lambda i: (0, i)),
        pl.BlockSpec((1, BR), lambda i: (0, i)),
        pl.BlockSpec((1, DO), lambda i: (0, 0)),
    ],
    out_specs=pl.BlockSpec((BR, DO), lambda i: (i, 0)),
    out_shape=jax.ShapeDtypeStruct((P, DO), jnp.float32),
)


def _k4_body(p_ref, nd_ref, wmu_ref, bmu_ref, wls_ref, bls_ref, eps_ref,
             mu_ref, sg_ref, z_ref):
    agg = (p_ref[0] + p_ref[1]) * nd_ref[0, :][:, None]
    mu = jnp.tanh(jnp.dot(agg, wmu_ref[...],
                          preferred_element_type=jnp.float32) + bmu_ref[0, :][None, :])
    ls = jnp.tanh(jnp.dot(agg, wls_ref[...],
                          preferred_element_type=jnp.float32) + bls_ref[0, :][None, :])
    sg = jnp.exp(ls)
    mu_ref[...] = mu
    sg_ref[...] = sg
    z_ref[...] = mu + sg * eps_ref[...]


_k4 = pl.pallas_call(
    _k4_body,
    grid=(G,),
    in_specs=[
        pl.BlockSpec((NC, BR, DO), lambda i: (0, i, 0)),
        pl.BlockSpec((1, BR), lambda i: (0, i)),
        pl.BlockSpec((DO, DO), lambda i: (0, 0)),
        pl.BlockSpec((1, DO), lambda i: (0, 0)),
        pl.BlockSpec((DO, DO), lambda i: (0, 0)),
        pl.BlockSpec((1, DO), lambda i: (0, 0)),
        pl.BlockSpec((BR, DO), lambda i: (i, 0)),
    ],
    out_specs=[
        pl.BlockSpec((BR, DO), lambda i: (i, 0)),
        pl.BlockSpec((BR, DO), lambda i: (i, 0)),
        pl.BlockSpec((BR, DO), lambda i: (i, 0)),
    ],
    out_shape=[
        jax.ShapeDtypeStruct((P, DO), jnp.float32),
        jax.ShapeDtypeStruct((P, DO), jnp.float32),
        jax.ShapeDtypeStruct((P, DO), jnp.float32),
    ],
)


def kernel(features, edge_index, W1, b1, W2, b2, Wmu, bmu, Wls, bls, eps):
    src = edge_index[0]
    dst = edge_index[1]
    ar = jnp.arange(EP - E, dtype=jnp.int32)
    pad_zero_rows = SENT + ar % (P - N)
    src2d = jnp.concatenate([src, pad_zero_rows]).reshape(EP // CHUNK, CHUNK)
    dstdeg2d = jnp.concatenate([dst, pad_zero_rows]).reshape(EP // CHUNK, CHUNK)
    dst2d = jnp.concatenate([dst, ar % N]).reshape(EP // CHUNK, CHUNK)

    x = jnp.zeros((P, DI), jnp.float32).at[:N].set(features)
    epsp = jnp.zeros((P, DO), jnp.float32).at[:N].set(eps)

    cnt = _deg(src2d, dstdeg2d)

    h0, ns, nd = _k1(x, W1, cnt, cnt)
    p1 = _prop(h0, src2d, dst2d)
    h1 = _k2(p1, ns, nd, W2, b1.reshape(1, DO))
    p2 = _prop(h1, src2d, dst2d)
    h2 = _k3(p2, ns, nd, b2.reshape(1, DO))
    p3 = _prop(h2, src2d, dst2d)
    mu, sg, z = _k4(p3, nd, Wmu, bmu.reshape(1, DO), Wls, bls.reshape(1, DO),
                    epsp)
    return mu[:N], sg[:N], z[:N]
